# Initial kernel scaffold; baseline (speedup 1.0000x reference)
import jax
import jax.numpy as jnp
from jax.experimental import pallas as pl


def kernel(query, queue_anchor, queue_label):
    raise NotImplementedError("write your pallas kernel here")


# speedup vs baseline: 2.6256x; 2.6256x over previous
import functools

import jax
import jax.numpy as jnp
from jax import lax
from jax.experimental import pallas as pl
from jax.experimental.pallas import tpu as pltpu
from jax.experimental.pallas import tpu_sc as plsc

TILE_K = 2048
CHUNK = 128
CHUNKS_PER_TILE = TILE_K // CHUNK
KNN = 8
IMAX = 2147483647

CCAP = 32
ECAP = 480


def _tc_body(q_ref, a_ref, dist_ref, mc_ref, *, n_valid):
    i = pl.program_id(0)
    a = a_ref[...]
    la = jnp.log(a)
    prod = a * la
    lq = jnp.log(q_ref[...])
    ones8 = jnp.ones((8, a.shape[1]), jnp.float32)
    rt = lax.dot_general(ones8, prod, (((1,), (1,)), ((), ())),
                         preferred_element_type=jnp.float32)[0:1]
    cross = lax.dot_general(lq, a, (((1,), (1,)), ((), ())),
                            preferred_element_type=jnp.float32)
    dist = (rt - cross) * (1.0 / a.shape[1])
    col = lax.broadcasted_iota(jnp.int32, dist.shape, 1)
    k = i * TILE_K + col
    dist = jnp.where(k < n_valid, dist, jnp.inf)
    dist_ref[...] = dist
    mc = jnp.min(dist.reshape(dist.shape[0], CHUNKS_PER_TILE, CHUNK), axis=2)
    mc_ref[...] = mc.reshape(1, dist.shape[0], CHUNKS_PER_TILE)


def _tc_distance(query, anchors):
    bsz, dim = query.shape
    kk = anchors.shape[0]
    ntiles = -(-kk // TILE_K)
    k_pad = ntiles * TILE_K
    return pl.pallas_call(
        functools.partial(_tc_body, n_valid=kk),
        grid=(ntiles,),
        in_specs=[
            pl.BlockSpec((bsz, dim), lambda i: (0, 0)),
            pl.BlockSpec((TILE_K, dim), lambda i: (i, 0)),
        ],
        out_specs=[
            pl.BlockSpec((bsz, TILE_K), lambda i: (0, i)),
            pl.BlockSpec((1, bsz, CHUNKS_PER_TILE), lambda i: (i, 0, 0)),
        ],
        out_shape=[
            jax.ShapeDtypeStruct((bsz, k_pad), jnp.float32),
            jax.ShapeDtypeStruct((ntiles, bsz, CHUNKS_PER_TILE), jnp.float32),
        ],
    )(query, anchors)


def _sc_select(dist1, mc1, labels, *, bsz, kk, ntiles, interpret=False):
    nchunk = ntiles * CHUNKS_PER_TILE
    nsteps = nchunk // 16
    try:
        info = plsc.get_sparse_core_info()
        nc, ns = info.num_cores, info.num_subcores
    except Exception:
        nc, ns = 2, 16
    nw = nc * ns
    qpw = bsz // nw
    mesh = plsc.VectorSubcoreMesh(core_axis_name="c", subcore_axis_name="s",
                                  num_cores=nc, num_subcores=ns)

    gdn = lax.GatherDimensionNumbers(offset_dims=(), collapsed_slice_dims=(0,),
                                     start_index_map=(0,))

    def body(dist_hbm, mc_hbm, lbl_hbm, out_hbm,
             minrow_v, cbuf_v, lblbuf_v, res_v, sel_sm, sem):
        cid = lax.axis_index("c")
        sid = lax.axis_index("s")
        wid = sid * nc + cid
        iot = lax.iota(jnp.int32, 16)
        inf16 = jnp.full((16,), jnp.inf, jnp.float32)

        def shuf(x, sh):
            return lax.gather(x, (iot ^ sh)[:, None], gdn, (1,),
                              mode=lax.GatherScatterMode.PROMISE_IN_BOUNDS)

        def bmin_pair(bv, bi):
            for sh in (8, 4, 2, 1):
                gv, gi = shuf(bv, sh), shuf(bi, sh)
                better = (gv < bv) | ((gv == bv) & (gi < bi))
                bv = jnp.where(better, gv, bv)
                bi = jnp.where(better, gi, bi)
            return bv, bi

        def bmin_triple(bv, bi, bs):
            for sh in (8, 4, 2, 1):
                gv, gi, gs = shuf(bv, sh), shuf(bi, sh), shuf(bs, sh)
                better = (gv < bv) | ((gv == bv) & (gi < bi))
                bv = jnp.where(better, gv, bv)
                bi = jnp.where(better, gi, bi)
                bs = jnp.where(better, gs, bs)
            return bv, bi, bs

        def per_query(q, res16):
            b = wid * qpw + q

            hs = []
            for t in range(nsteps):
                hs.append(pltpu.async_copy(
                    mc_hbm.at[pl.ds((t * bsz + b) * 16, 16)],
                    minrow_v.at[pl.ds(t * 16, 16)], sem))
            for h in hs:
                h.wait()

            minis = inf16
            cv = jnp.full((16,), IMAX, jnp.int32)
            for p in range(KNN):
                def scan(t, carry):
                    av, ai = carry
                    v = minrow_v[pl.ds(t * 16, 16)]
                    idx = t * 16 + iot
                    better = v < av
                    return (jnp.where(better, v, av),
                            jnp.where(better, idx, ai))
                av, ai = lax.fori_loop(0, nsteps, scan,
                                       (inf16, jnp.full((16,), IMAX, jnp.int32)))
                av, ai = bmin_pair(av, ai)
                mstar, cstar = av[0], ai[0]
                minis = jnp.where(iot == p, mstar, minis)
                cv = jnp.where(iot == p, cstar, cv)
                g = cstar // 16
                r = cstar - g * 16
                row = minrow_v[pl.ds(g * 16, 16)]
                minrow_v[pl.ds(g * 16, 16)] = jnp.where(iot == r, jnp.inf, row)
                sel_sm[8 + p] = cstar

            hs = []
            for j in range(KNN):
                cj = sel_sm[8 + j]
                hs.append(pltpu.async_copy(
                    dist_hbm.at[pl.ds((b * nchunk + cj) * CHUNK, CHUNK)],
                    cbuf_v.at[pl.ds(j * CHUNK, CHUNK)], sem))
            for h in hs:
                h.wait()

            for p in range(KNN):
                _, cwin, slot = bmin_triple(minis, cv, iot)
                slot0 = slot[0]
                cwin0 = cwin[0]
                base = slot0 * CHUNK

                def cscan(s, carry):
                    av, ai = carry
                    v = cbuf_v[pl.ds(base + s * 16, 16)]
                    idx = s * 16 + iot
                    better = v < av
                    return (jnp.where(better, v, av),
                            jnp.where(better, idx, ai))
                av, ai = lax.fori_loop(0, CHUNK // 16, cscan,
                                       (inf16, jnp.full((16,), IMAX, jnp.int32)))
                av, ai = bmin_pair(av, ai)
                jstar = ai[0]
                kstar = cwin0 * CHUNK + jstar
                sel_sm[p] = kstar

                g = jstar // 16
                r = jstar - g * 16
                row = cbuf_v[pl.ds(base + g * 16, 16)]
                cbuf_v[pl.ds(base + g * 16, 16)] = jnp.where(iot == r, jnp.inf, row)

                def mscan(s, a):
                    return jnp.minimum(a, cbuf_v[pl.ds(base + s * 16, 16)])
                a = lax.fori_loop(0, CHUNK // 16, mscan, inf16)
                for sh in (8, 4, 2, 1):
                    a = jnp.minimum(a, shuf(a, sh))
                minis = jnp.where(iot == slot0, a[0], minis)

            hs = []
            for j in range(KNN):
                kj = sel_sm[j]
                w0 = (kj // 16) * 16
                sel_sm[8 + j] = kj - w0
                hs.append(pltpu.async_copy(lbl_hbm.at[pl.ds(w0, 16)],
                                           lblbuf_v.at[pl.ds(j * 32, 16)], sem))
            for h in hs:
                h.wait()
            vote = jnp.int32(0)
            for j in range(KNN):
                rj = sel_sm[8 + j]
                wl = lblbuf_v[pl.ds(j * 32 + rj, 16)]
                vote = vote + wl[0]
            resq = jnp.where(vote >= 5, 1, 0).astype(jnp.int32)
            return jnp.where(iot == q, resq, res16)

        res16 = lax.fori_loop(0, qpw, per_query, jnp.zeros((16,), jnp.int32))
        res_v[...] = res16
        pltpu.sync_copy(res_v, out_hbm.at[pl.ds(wid * 16, 16)])

    return pl.kernel(
        body,
        out_type=jax.ShapeDtypeStruct((nw * 16,), jnp.int32),
        mesh=mesh,
        scratch_types=[
            pltpu.VMEM((nchunk,), jnp.float32),
            pltpu.VMEM((KNN * CHUNK,), jnp.float32),
            pltpu.VMEM((KNN * 32,), jnp.int32),
            pltpu.VMEM((16,), jnp.int32),
            pltpu.SMEM((16,), jnp.int32),
            pltpu.SemaphoreType.DMA,
        ],
        interpret=interpret,
    )(dist1, mc1, labels)


def kernel(query, queue_anchor, queue_label):
    bsz = query.shape[0]
    kk = queue_anchor.shape[0]
    labels = queue_label.astype(jnp.int32)
    dist, mc = _tc_distance(query, queue_anchor)
    ntiles = mc.shape[0]
    res = _sc_select(dist.reshape(-1), mc.reshape(-1), labels,
                     bsz=bsz, kk=kk, ntiles=ntiles)
    nw = res.shape[0] // 16
    return res.reshape(nw, 16)[:, : bsz // nw].reshape(bsz)
